# baseline (speedup 1.0000x reference)
import functools

import jax
import jax.numpy as jnp
from jax import lax
from jax.experimental import pallas as pl
from jax.experimental.pallas import tpu as pltpu
from jax.experimental.pallas import tpu_sc as plsc

_M, _K, _D = 8, 8192, 32
_N, _H, _W = 16, 32, 32
_HW = _H * _W
_KH, _KL = _K // 128, 128
_MC = _M * _D
_NW = 32
_CPT = _MC // _NW


def _pcb_body(wq_ref, cb_ref, bq_ref, out_ref):
    w = wq_ref[0]
    cb = cb_ref[0]
    out_ref[...] = lax.dot_general(
        w, cb, (((1,), (1,)), ((), ())),
        preferred_element_type=jnp.float32) + bq_ref[0]


def _compute_pcb(codebook, wq, bq):
    return pl.pallas_call(
        _pcb_body,
        grid=(_M,),
        in_specs=[
            pl.BlockSpec((1, _D, _D), lambda m: (m, 0, 0)),
            pl.BlockSpec((1, _K, _D), lambda m: (m, 0, 0)),
            pl.BlockSpec((1, _D, 1), lambda m: (m, 0, 0)),
        ],
        out_specs=pl.BlockSpec((_D, _K), lambda m: (m, 0)),
        out_shape=jax.ShapeDtypeStruct((_MC, _K), jnp.float32),
    )(wq, codebook, bq.reshape(_M, _D, 1))


def _decode_sc(pcb, idx, nimg):
    info = plsc.get_sparse_core_info()
    nc = info.num_cores
    mesh = plsc.VectorSubcoreMesh(core_axis_name="c", subcore_axis_name="s")

    @functools.partial(
        pl.kernel,
        mesh=mesh,
        compiler_params=pltpu.CompilerParams(use_tc_tiling_on_sc=True,
                                             needs_layout_passes=False),
        out_type=jax.ShapeDtypeStruct((nimg, _MC, _HW), jnp.float32),
        scratch_types=[
            pltpu.VMEM((_CPT, _K), jnp.float32),
            pltpu.VMEM((1, 1, _HW), jnp.int32),
            pltpu.VMEM((1, 1, _HW), jnp.int32),
            pltpu.VMEM((1, _CPT, _HW), jnp.float32),
            pltpu.VMEM((1, _CPT, _HW), jnp.float32),
            pltpu.SemaphoreType.DMA,
            pltpu.SemaphoreType.DMA,
            pltpu.SemaphoreType.DMA,
            pltpu.SemaphoreType.DMA,
        ],
    )
    def k(pcb_hbm, idx_hbm, out_hbm, table_v, idx_a, idx_b, ob_a, ob_b,
          is_a, is_b, os_a, os_b):
        wid = lax.axis_index("s") * nc + lax.axis_index("c")
        m = wid // 4
        row0 = pl.multiple_of(wid * _CPT, _CPT)
        pltpu.sync_copy(pcb_hbm.at[pl.ds(row0, _CPT)], table_v)

        csplat = [jnp.full((16,), c, jnp.int32) for c in range(_CPT)]

        def gather_image(idx_v, obuf):
            for q in range(_HW // 16):
                codev = idx_v[0, 0, pl.ds(q * 16, 16)]
                vals = [plsc.load_gather(table_v, [csplat[c], codev])
                        for c in range(_CPT)]
                for c in range(_CPT):
                    obuf[0, c, pl.ds(q * 16, 16)] = vals[c]

        def idx_cp(n, idx_v, sem):
            return pltpu.make_async_copy(
                idx_hbm.at[pl.ds(n, 1), pl.ds(m, 1)], idx_v, sem)

        def out_cp(n, obuf, sem):
            return pltpu.make_async_copy(
                obuf, out_hbm.at[pl.ds(n, 1), pl.ds(row0, _CPT)], sem)

        idx_cp(0, idx_a, is_a).start()

        @pl.loop(0, nimg // 2)
        def _pair_loop(i):
            n0 = i * 2
            idx_cp(n0, idx_a, is_a).wait()
            idx_cp(n0 + 1, idx_b, is_b).start()

            @pl.when(i > 0)
            def _():
                out_cp(n0 - 2, ob_a, os_a).wait()

            gather_image(idx_a, ob_a)
            out_cp(n0, ob_a, os_a).start()

            @pl.when(i < nimg // 2 - 1)
            def _():
                idx_cp(n0 + 2, idx_a, is_a).start()

            idx_cp(n0 + 1, idx_b, is_b).wait()

            @pl.when(i > 0)
            def _():
                out_cp(n0 - 1, ob_b, os_b).wait()

            gather_image(idx_b, ob_b)
            out_cp(n0 + 1, ob_b, os_b).start()

        out_cp(nimg - 2, ob_a, os_a).wait()
        out_cp(nimg - 1, ob_b, os_b).wait()

    return k(pcb, idx)


def kernel(codes, codebook, wq, bq):
    pcb = _compute_pcb(codebook, wq, bq)
    nh = _N // 2
    idx_a = codes[:nh].transpose(0, 3, 1, 2).reshape(nh, _M, _HW)
    idx_b = codes[nh:].transpose(0, 3, 1, 2).reshape(nh, _M, _HW)
    out_a = _decode_sc(pcb, idx_a, nh).reshape(nh, _MC, _H, _W)
    out_b = _decode_sc(pcb, idx_b, nh).reshape(nh, _MC, _H, _W)
    return jnp.concatenate([out_a, out_b], axis=0)


# speedup vs baseline: 1.1738x; 1.1738x over previous
import functools

import jax
import jax.numpy as jnp
from jax import lax
from jax.experimental import pallas as pl
from jax.experimental.pallas import tpu as pltpu
from jax.experimental.pallas import tpu_sc as plsc

_M, _K, _D = 8, 8192, 32
_N, _H, _W = 16, 32, 32
_HW = _H * _W
_KH, _KL = _K // 128, 128
_MC = _M * _D
_NW = 32
_CPT = _MC // _NW


def _pcb_body(wq_ref, cb_ref, bq_ref, out_ref):
    w = wq_ref[0]
    cb = cb_ref[0]
    out_ref[...] = lax.dot_general(
        w, cb, (((1,), (1,)), ((), ())),
        preferred_element_type=jnp.float32) + bq_ref[0]


def _compute_pcb(codebook, wq, bq):
    return pl.pallas_call(
        _pcb_body,
        grid=(_M,),
        in_specs=[
            pl.BlockSpec((1, _D, _D), lambda m: (m, 0, 0)),
            pl.BlockSpec((1, _K, _D), lambda m: (m, 0, 0)),
            pl.BlockSpec((1, _D, 1), lambda m: (m, 0, 0)),
        ],
        out_specs=pl.BlockSpec((_D, _K), lambda m: (m, 0)),
        out_shape=jax.ShapeDtypeStruct((_MC, _K), jnp.float32),
    )(wq, codebook, bq.reshape(_M, _D, 1))


def _decode_sc(pcb, idx, nimg):
    info = plsc.get_sparse_core_info()
    nc = info.num_cores
    mesh = plsc.VectorSubcoreMesh(core_axis_name="c", subcore_axis_name="s")

    @functools.partial(
        pl.kernel,
        mesh=mesh,
        compiler_params=pltpu.CompilerParams(use_tc_tiling_on_sc=True,
                                             needs_layout_passes=False),
        out_type=jax.ShapeDtypeStruct((nimg, _MC, _HW), jnp.float32),
        scratch_types=[
            pltpu.VMEM((_CPT, _K), jnp.float32),
            pltpu.VMEM((1, 1, _HW), jnp.int32),
            pltpu.VMEM((1, 1, _HW), jnp.int32),
            pltpu.VMEM((1, _CPT, _HW), jnp.float32),
            pltpu.VMEM((1, _CPT, _HW), jnp.float32),
            pltpu.SemaphoreType.DMA,
            pltpu.SemaphoreType.DMA,
            pltpu.SemaphoreType.DMA,
            pltpu.SemaphoreType.DMA,
        ],
    )
    def k(pcb_hbm, idx_hbm, out_hbm, table_v, idx_a, idx_b, ob_a, ob_b,
          is_a, is_b, os_a, os_b):
        wid = lax.axis_index("s") * nc + lax.axis_index("c")
        m = wid // 4
        row0 = pl.multiple_of(wid * _CPT, _CPT)
        pltpu.sync_copy(pcb_hbm.at[pl.ds(row0, _CPT)], table_v)

        csplat = [jnp.full((16,), c, jnp.int32) for c in range(_CPT)]

        def gather_image(idx_v, obuf):
            for q in range(_HW // 16):
                codev = idx_v[0, 0, pl.ds(q * 16, 16)]
                vals = [plsc.load_gather(table_v, [csplat[c], codev])
                        for c in range(_CPT)]
                for c in range(_CPT):
                    obuf[0, c, pl.ds(q * 16, 16)] = vals[c]

        def idx_cp(n, idx_v, sem):
            return pltpu.make_async_copy(
                idx_hbm.at[pl.ds(n, 1), pl.ds(m, 1)], idx_v, sem)

        def out_cp(n, obuf, sem):
            return pltpu.make_async_copy(
                obuf, out_hbm.at[pl.ds(n, 1), pl.ds(row0, _CPT)], sem)

        idx_cp(0, idx_a, is_a).start()

        @pl.loop(0, nimg // 2)
        def _pair_loop(i):
            n0 = i * 2
            idx_cp(n0, idx_a, is_a).wait()
            idx_cp(n0 + 1, idx_b, is_b).start()

            @pl.when(i > 0)
            def _():
                out_cp(n0 - 2, ob_a, os_a).wait()

            gather_image(idx_a, ob_a)
            out_cp(n0, ob_a, os_a).start()

            @pl.when(i < nimg // 2 - 1)
            def _():
                idx_cp(n0 + 2, idx_a, is_a).start()

            idx_cp(n0 + 1, idx_b, is_b).wait()

            @pl.when(i > 0)
            def _():
                out_cp(n0 - 1, ob_b, os_b).wait()

            gather_image(idx_b, ob_b)
            out_cp(n0 + 1, ob_b, os_b).start()

        out_cp(nimg - 2, ob_a, os_a).wait()
        out_cp(nimg - 1, ob_b, os_b).wait()

    return k(pcb, idx)


def kernel(codes, codebook, wq, bq):
    pcb = _compute_pcb(codebook, wq, bq)
    idx = codes.transpose(0, 3, 1, 2).reshape(_N, _M, _HW)
    return _decode_sc(pcb, idx, _N).reshape(_N, _MC, _H, _W)
